# baseline (speedup 1.0000x reference)
import functools

import jax
import jax.numpy as jnp
from jax import lax
from jax.experimental import pallas as pl
from jax.experimental.pallas import tpu as pltpu, tpu_sc as plsc

_E_TOTAL = 2 * 160000
_NW = 32
_PER_W = _E_TOTAL // _NW
_VECS = _PER_W // 16


def _pool_body(x_ref, o1, o2, o3, o4):
    v = x_ref[...]

    def pool(t, groups):
        even = jnp.concatenate(
            [t[..., (2 * j) * 256:(2 * j + 1) * 256] for j in range(groups)],
            axis=-1)
        odd = jnp.concatenate(
            [t[..., (2 * j + 1) * 256:(2 * j + 2) * 256] for j in range(groups)],
            axis=-1)
        return (even + odd) * 0.5

    p1 = pool(v, 8)
    p2 = pool(p1, 4)
    p3 = pool(p2, 2)
    p4 = pool(p3, 1)
    o1[...] = p1
    o2[...] = p2
    o3[...] = p3
    o4[...] = p4


_sc_mesh = plsc.VectorSubcoreMesh(core_axis_name="c", subcore_axis_name="s")


@functools.partial(
    pl.kernel,
    mesh=_sc_mesh,
    out_type=[jax.ShapeDtypeStruct((_E_TOTAL,), jnp.int32)] * 4,
    scratch_types=[pltpu.VMEM((_PER_W,), jnp.int32) for _ in range(5)],
)
def _edge_sc(e_hbm, o1_hbm, o2_hbm, o3_hbm, o4_hbm, buf, b1, b2, b3, b4):
    wid = lax.axis_index("s") * 2 + lax.axis_index("c")
    base = wid * _PER_W
    pltpu.sync_copy(e_hbm.at[pl.ds(base, _PER_W)], buf)

    def body(i, carry):
        sl = pl.ds(i * 16, 16)
        v = buf[sl]
        b1[sl] = v >> 1
        b2[sl] = v >> 2
        b3[sl] = v >> 3
        b4[sl] = v >> 4
        return carry

    lax.fori_loop(0, _VECS, body, 0)
    pltpu.sync_copy(b1, o1_hbm.at[pl.ds(base, _PER_W)])
    pltpu.sync_copy(b2, o2_hbm.at[pl.ds(base, _PER_W)])
    pltpu.sync_copy(b3, o3_hbm.at[pl.ds(base, _PER_W)])
    pltpu.sync_copy(b4, o4_hbm.at[pl.ds(base, _PER_W)])


def kernel(x, edge_index):
    xr = x.reshape(5, 125, 4096)

    fspec = lambda shp: pl.BlockSpec((1,) + shp[1:], lambda i: (i, 0, 0))
    p1, p2, p3, p4 = pl.pallas_call(
        _pool_body,
        grid=(5,),
        in_specs=[fspec((5, 125, 4096))],
        out_specs=[
            fspec((5, 125, 2048)), fspec((5, 125, 1024)),
            fspec((5, 125, 512)), fspec((5, 125, 256)),
        ],
        out_shape=[
            jax.ShapeDtypeStruct((5, 125, 2048), jnp.float32),
            jax.ShapeDtypeStruct((5, 125, 1024), jnp.float32),
            jax.ShapeDtypeStruct((5, 125, 512), jnp.float32),
            jax.ShapeDtypeStruct((5, 125, 256), jnp.float32),
        ],
    )(xr)

    f1, f2, f3, f4 = _edge_sc(edge_index.reshape(_E_TOTAL))

    x1 = p1.reshape(5000, 256)
    x2 = p2.reshape(2500, 256)
    x3 = p3.reshape(1250, 256)
    x4 = p4.reshape(625, 256)
    e1 = f1.reshape(2, 160000)
    e2 = f2.reshape(2, 160000)
    e3 = f3.reshape(2, 160000)
    e4 = f4.reshape(2, 160000)
    return (x, x1, x2, x3, x4, edge_index, e1, e2, e3, e4)


# speedup vs baseline: 1.1071x; 1.1071x over previous
import functools

import jax
import jax.numpy as jnp
from jax import lax
from jax.experimental import pallas as pl
from jax.experimental.pallas import tpu as pltpu, tpu_sc as plsc

_E_TOTAL = 2 * 160000
_NW = 32
_PER_W = _E_TOTAL // _NW
_VECS = _PER_W // 16


def _pool_body(x_ref, e_ref, o0, o1, o2, o3, o4, g0, g1, g2, g3, g4):
    v = x_ref[...]
    o0[...] = v
    e = e_ref[...]
    g0[...] = e
    g1[...] = e >> 1
    g2[...] = e >> 2
    g3[...] = e >> 3
    g4[...] = e >> 4

    def pool(t, groups):
        even = jnp.concatenate(
            [t[..., (2 * j) * 256:(2 * j + 1) * 256] for j in range(groups)],
            axis=-1)
        odd = jnp.concatenate(
            [t[..., (2 * j + 1) * 256:(2 * j + 2) * 256] for j in range(groups)],
            axis=-1)
        return (even + odd) * 0.5

    p1 = pool(v, 8)
    p2 = pool(p1, 4)
    p3 = pool(p2, 2)
    p4 = pool(p3, 1)
    o1[...] = p1
    o2[...] = p2
    o3[...] = p3
    o4[...] = p4


def kernel(x, edge_index):
    xr = x.reshape(5, 125, 4096)
    er = edge_index.reshape(5, 250, 256)

    fspec = lambda shp: pl.BlockSpec((1,) + shp[1:], lambda i: (i, 0, 0))
    outs = pl.pallas_call(
        _pool_body,
        grid=(5,),
        in_specs=[fspec((5, 125, 4096)), fspec((5, 250, 256))],
        out_specs=[
            fspec((5, 125, 4096)),
            fspec((5, 125, 2048)), fspec((5, 125, 1024)),
            fspec((5, 125, 512)), fspec((5, 125, 256)),
            fspec((5, 250, 256)), fspec((5, 250, 256)),
            fspec((5, 250, 256)), fspec((5, 250, 256)),
            fspec((5, 250, 256)),
        ],
        out_shape=[
            jax.ShapeDtypeStruct((5, 125, 4096), jnp.float32),
            jax.ShapeDtypeStruct((5, 125, 2048), jnp.float32),
            jax.ShapeDtypeStruct((5, 125, 1024), jnp.float32),
            jax.ShapeDtypeStruct((5, 125, 512), jnp.float32),
            jax.ShapeDtypeStruct((5, 125, 256), jnp.float32),
            jax.ShapeDtypeStruct((5, 250, 256), jnp.int32),
            jax.ShapeDtypeStruct((5, 250, 256), jnp.int32),
            jax.ShapeDtypeStruct((5, 250, 256), jnp.int32),
            jax.ShapeDtypeStruct((5, 250, 256), jnp.int32),
            jax.ShapeDtypeStruct((5, 250, 256), jnp.int32),
        ],
    )(xr, er)
    p0, p1, p2, p3, p4, g0, g1, g2, g3, g4 = outs

    x0 = p0.reshape(10000, 256)
    x1 = p1.reshape(5000, 256)
    x2 = p2.reshape(2500, 256)
    x3 = p3.reshape(1250, 256)
    x4 = p4.reshape(625, 256)
    e0 = g0.reshape(2, 160000)
    e1 = g1.reshape(2, 160000)
    e2 = g2.reshape(2, 160000)
    e3 = g3.reshape(2, 160000)
    e4 = g4.reshape(2, 160000)
    return (x0, x1, x2, x3, x4, e0, e1, e2, e3, e4)
